# Initial kernel scaffold; baseline (speedup 1.0000x reference)
import jax
import jax.numpy as jnp
from jax.experimental import pallas as pl


def kernel(x, edge_index, batch, W1, b1, W2, b2, Wfc, bfc):
    raise NotImplementedError("write your pallas kernel here")


# speedup vs baseline: 13.0986x; 13.0986x over previous
import functools

import jax
import jax.numpy as jnp
from jax import lax
from jax.experimental import pallas as pl
from jax.experimental.pallas import tpu as pltpu
from jax.experimental.pallas import tpu_sc as plsc

N = 10000
E = 320000
D = 128
G = 128
NCLS = 10
NP = 10240

NTILE = 16
NWORK = 32
EPW = E // NWORK
CH = 80
NCH = EPW // CH
RPT = NP // NTILE

_mesh = plsc.VectorSubcoreMesh(core_axis_name="c", subcore_axis_name="s")


@functools.partial(
    pl.kernel,
    mesh=_mesh,
    out_type=jax.ShapeDtypeStruct((2, NP), jnp.float32),
    scratch_types=[
        pltpu.VMEM((CH,), jnp.int32),
        pltpu.VMEM((CH,), jnp.float32),
        pltpu.VMEM((RPT,), jnp.float32),
        pltpu.VMEM_SHARED((NP,), jnp.float32),
    ],
)
def _deg_kernel(dst_hbm, out_hbm, didx_v, ones_v, zeros_v, acc_sh):
    cid = lax.axis_index("c")
    sid = lax.axis_index("s")

    def _fill(ref, n, val):
        def body(j, c):
            ref[pl.ds(j * 16, 16)] = jnp.full((16,), val, jnp.float32)
            return c
        lax.fori_loop(0, n // 16, body, 0)

    _fill(ones_v, CH, 1.0)
    _fill(zeros_v, RPT, 0.0)

    r0 = sid * RPT
    pltpu.sync_copy(zeros_v, acc_sh.at[pl.ds(r0, RPT)])
    plsc.subcore_barrier()

    e0 = (cid * NTILE + sid) * EPW

    def body(i, c):
        pltpu.sync_copy(dst_hbm.at[pl.ds(e0 + i * CH, CH)], didx_v)
        pltpu.sync_copy(ones_v, acc_sh.at[didx_v], add=True)
        return c

    lax.fori_loop(0, NCH, body, 0)
    plsc.subcore_barrier()
    pltpu.sync_copy(acc_sh.at[pl.ds(r0, RPT)], out_hbm.at[cid, pl.ds(r0, RPT)])


@functools.partial(
    pl.kernel,
    mesh=_mesh,
    out_type=jax.ShapeDtypeStruct((2, NP, D), jnp.float32),
    scratch_types=[
        pltpu.VMEM((CH,), jnp.int32),
        pltpu.VMEM((CH,), jnp.int32),
        pltpu.VMEM((CH, D), jnp.float32),
        pltpu.VMEM_SHARED((NP, D), jnp.float32),
        pltpu.SemaphoreType.DMA,
    ],
)
def _agg_kernel(hh_hbm, src_hbm, dst_hbm, out_hbm, sidx_v, didx_v, rows_v,
                acc_sh, sem):
    cid = lax.axis_index("c")
    sid = lax.axis_index("s")

    r0 = sid * RPT
    pltpu.sync_copy(hh_hbm.at[pl.ds(r0, RPT)], acc_sh.at[pl.ds(r0, RPT)])
    plsc.subcore_barrier()

    e0 = (cid * NTILE + sid) * EPW

    def body(i, c):
        base = e0 + i * CH
        pltpu.sync_copy(src_hbm.at[pl.ds(base, CH)], sidx_v)
        pltpu.sync_copy(dst_hbm.at[pl.ds(base, CH)], didx_v)
        pltpu.async_copy(hh_hbm.at[sidx_v], rows_v, sem).wait()
        pltpu.sync_copy(rows_v, acc_sh.at[didx_v], add=True)
        return c

    lax.fori_loop(0, NCH, body, 0)
    plsc.subcore_barrier()
    pltpu.sync_copy(acc_sh.at[pl.ds(r0, RPT)],
                    out_hbm.at[cid, pl.ds(r0, RPT)])


BR = 1280


def _lin1_body(x_ref, w_ref, deg_ref, out_ref):
    h = jnp.dot(x_ref[...], w_ref[...], preferred_element_type=jnp.float32)
    out_ref[...] = h * lax.rsqrt(deg_ref[...])


def _lin1(x_pad, W1, deg_col):
    return pl.pallas_call(
        _lin1_body,
        grid=(NP // BR,),
        in_specs=[
            pl.BlockSpec((BR, D), lambda i: (i, 0)),
            pl.BlockSpec((D, D), lambda i: (0, 0)),
            pl.BlockSpec((BR, 1), lambda i: (i, 0)),
        ],
        out_specs=pl.BlockSpec((BR, D), lambda i: (i, 0)),
        out_shape=jax.ShapeDtypeStruct((NP, D), jnp.float32),
    )(x_pad, W1, deg_col)


def _mid_body(p0_ref, p1_ref, hh_ref, deg_ref, b_ref, w_ref, out_ref):
    s = p0_ref[...] + p1_ref[...] - hh_ref[...]
    dinv = lax.rsqrt(deg_ref[...])
    h2 = jnp.maximum(dinv * s + b_ref[...], 0.0)
    out_ref[...] = jnp.dot(
        h2, w_ref[...], preferred_element_type=jnp.float32) * dinv


def _mid(p0, p1, hh1, deg_col, b1_row, W2):
    return pl.pallas_call(
        _mid_body,
        grid=(NP // BR,),
        in_specs=[
            pl.BlockSpec((BR, D), lambda i: (i, 0)),
            pl.BlockSpec((BR, D), lambda i: (i, 0)),
            pl.BlockSpec((BR, D), lambda i: (i, 0)),
            pl.BlockSpec((BR, 1), lambda i: (i, 0)),
            pl.BlockSpec((1, D), lambda i: (0, 0)),
            pl.BlockSpec((D, D), lambda i: (0, 0)),
        ],
        out_specs=pl.BlockSpec((BR, D), lambda i: (i, 0)),
        out_shape=jax.ShapeDtypeStruct((NP, D), jnp.float32),
    )(p0, p1, hh1, deg_col, b1_row, W2)


def _final_body(q0_ref, q1_ref, hh_ref, deg_ref, b_ref, batch_ref, wfc_ref,
                bfc_ref, out_ref):
    s = q0_ref[...] + q1_ref[...] - hh_ref[...]
    dinv = lax.rsqrt(deg_ref[...])
    t = jnp.maximum(dinv * s + b_ref[...], 0.0)
    g_iota = lax.broadcasted_iota(jnp.int32, (G, NP), 0)
    onehot_t = (batch_ref[...] == g_iota).astype(jnp.float32)
    sums = jnp.dot(onehot_t, t, preferred_element_type=jnp.float32)
    counts = jnp.dot(onehot_t, jnp.ones((NP, 1), jnp.float32),
                     preferred_element_type=jnp.float32)
    emb = sums / jnp.maximum(counts, 1.0)
    out_ref[...] = jnp.dot(
        emb, wfc_ref[...], preferred_element_type=jnp.float32) + bfc_ref[...]


def _final(q0, q1, hh2, deg_col, b2_row, batch_row, Wfc_pad, bfc_row):
    return pl.pallas_call(
        _final_body,
        out_shape=jax.ShapeDtypeStruct((G, G), jnp.float32),
    )(q0, q1, hh2, deg_col, b2_row, batch_row, Wfc_pad, bfc_row)


def kernel(x, edge_index, batch, W1, b1, W2, b2, Wfc, bfc):
    src = edge_index[0].astype(jnp.int32)
    dst = edge_index[1].astype(jnp.int32)
    x_pad = jnp.zeros((NP, D), jnp.float32).at[:N].set(x)
    batch_row = jnp.full((1, NP), -1, jnp.int32).at[0, :N].set(
        batch.astype(jnp.int32))

    degp = _deg_kernel(dst)
    deg_col = (1.0 + degp[0] + degp[1])[:, None]

    hh1 = _lin1(x_pad, W1, deg_col)
    p = _agg_kernel(hh1, src, dst)
    hh2 = _mid(p[0], p[1], hh1, deg_col, b1[None, :], W2)
    q = _agg_kernel(hh2, src, dst)

    Wfc_pad = jnp.zeros((D, G), jnp.float32).at[:, :NCLS].set(Wfc)
    bfc_row = jnp.zeros((1, G), jnp.float32).at[0, :NCLS].set(bfc)
    logits_pad = _final(q[0], q[1], hh2, deg_col, b2[None, :], batch_row,
                        Wfc_pad, bfc_row)
    return logits_pad[:, :NCLS]
